# baseline (speedup 1.0000x reference)
import jax
import jax.numpy as jnp
from jax import lax
from jax.experimental import pallas as pl
from jax.experimental.pallas import tpu as pltpu

_NBUF = 8
_CHUNK = 25
_NROWS = 1000
_BS = 64
_D = 256


def _pos_body(t_ref, o_ref, buf_ref, sems):
    t = t_ref[...]
    nchunks = _NROWS // _CHUNK

    def dma(c):
        slot = c % _NBUF
        return pltpu.make_async_copy(
            buf_ref.at[slot],
            o_ref.at[pl.ds(c * _CHUNK, _CHUNK)],
            sems.at[slot],
        )

    for c in range(nchunks):
        slot = c % _NBUF
        if c >= _NBUF:
            dma(c - _NBUF).wait()
        base = c * _CHUNK
        rids = base + lax.broadcasted_iota(jnp.int32, (_CHUNK, 32), 0)
        cids = lax.broadcasted_iota(jnp.int32, (_CHUNK, 32), 1)
        sel = (cids == rids % 10)
        sel |= (cids == 10 + (rids // 10) % 10)
        sel |= (cids == 20 + rids // 100)
        pos = jax.lax.dot_general(
            sel.astype(jnp.float32), t,
            dimension_numbers=(((1,), (0,)), ((), ())),
            preferred_element_type=jnp.float32,
            precision=jax.lax.Precision.HIGHEST,
        )
        buf_ref[slot] = jnp.broadcast_to(pos[:, None, :], (_CHUNK, _BS, _D))
        dma(c).start()

    for c in range(nchunks - _NBUF, nchunks):
        dma(c).wait()


def kernel(x, row_embed, col_embed, time_embed):
    bs, frame_num, h, w = x.shape[:4]
    d4 = row_embed.shape[1]
    d2 = time_embed.shape[1]
    d = 2 * d4 + d2
    n = frame_num * h * w

    t = jnp.zeros((32, d), jnp.float32)
    t = t.at[0:10, 0:d4].set(row_embed)
    t = t.at[10:20, d4:2 * d4].set(col_embed)
    t = t.at[20:30, 2 * d4:d].set(time_embed)

    out = pl.pallas_call(
        _pos_body,
        in_specs=[pl.BlockSpec(memory_space=pltpu.MemorySpace.VMEM)],
        out_specs=pl.BlockSpec(memory_space=pltpu.MemorySpace.HBM),
        out_shape=jax.ShapeDtypeStruct((n, bs, d), jnp.float32),
        scratch_shapes=[
            pltpu.VMEM((_NBUF, _CHUNK, bs, d), jnp.float32),
            pltpu.SemaphoreType.DMA((_NBUF,)),
        ],
    )(t)
    out = out.reshape(frame_num, h, w, bs, d)
    return jnp.transpose(out, (3, 0, 1, 2, 4))
